# baseline (speedup 1.0000x reference)
import functools

import jax
import jax.numpy as jnp
from jax import lax
from jax.experimental import pallas as pl
from jax.experimental.pallas import tpu as pltpu
from jax.experimental.pallas import tpu_sc as plsc

N = 25000
D = 128
E = 300000
NS = 16
NW = 32
C = 32
K = 294
EPAD = NW * C * K
NPAD = 25088
RPT = NPAD // NS
NB = 16
BR = NPAD // NB
NG = 21
GC = K // NG
CW = 32

assert K == NG * GC and EPAD >= E


def _prep_body(x_ref, g_ref, b_ref, wrel_ref, wself_ref, t_ref, s_ref):
    x = x_ref[...]
    mu = jnp.mean(x, axis=-1, keepdims=True)
    var = jnp.mean((x - mu) ** 2, axis=-1, keepdims=True)
    y = jnp.maximum((x - mu) * lax.rsqrt(var + 1e-5) * g_ref[0] + b_ref[0], 0.0)
    t_ref[...] = jnp.dot(y, wrel_ref[...], preferred_element_type=jnp.float32,
                         precision=lax.Precision.HIGHEST).astype(jnp.bfloat16)
    s_ref[...] = jnp.dot(y, wself_ref[...],
                         preferred_element_type=jnp.float32,
                         precision=lax.Precision.HIGHEST)


def _prep(x, g, b, wrel, wself):
    return pl.pallas_call(
        _prep_body,
        grid=(NB,),
        in_specs=[
            pl.BlockSpec((BR, D), lambda i: (i, 0)),
            pl.BlockSpec((1, D), lambda i: (0, 0)),
            pl.BlockSpec((1, D), lambda i: (0, 0)),
            pl.BlockSpec((D, D), lambda i: (0, 0)),
            pl.BlockSpec((D, D), lambda i: (0, 0)),
        ],
        out_specs=[
            pl.BlockSpec((BR, D), lambda i: (i, 0)),
            pl.BlockSpec((BR, D), lambda i: (i, 0)),
        ],
        out_shape=[jax.ShapeDtypeStruct((NPAD, D), jnp.bfloat16),
                   jax.ShapeDtypeStruct((N, D), jnp.float32)],
    )(x, g, b, wrel, wself)


@functools.cache
def _build_sc_agg():
  mesh = plsc.VectorSubcoreMesh(core_axis_name="c", subcore_axis_name="s")

  @functools.partial(
    pl.kernel,
    out_type=[
        jax.ShapeDtypeStruct((2, NPAD, D), jnp.bfloat16),
        jax.ShapeDtypeStruct((2, NPAD, CW), jnp.bfloat16),
        jax.ShapeDtypeStruct((2, NPAD, D), jnp.bfloat16),
        jax.ShapeDtypeStruct((2, NPAD, CW), jnp.bfloat16),
    ],
    mesh=mesh,
    compiler_params=pltpu.CompilerParams(use_tc_tiling_on_sc=False),
    scratch_types=[
        pltpu.VMEM_SHARED((NPAD, D), jnp.bfloat16),
        pltpu.VMEM_SHARED((NPAD, CW), jnp.bfloat16),
        pltpu.VMEM((GC, C), jnp.int32),
        pltpu.VMEM((GC, C), jnp.int32),
        pltpu.VMEM((2, C, D), jnp.bfloat16),
        pltpu.VMEM((C, CW), jnp.bfloat16),
        pltpu.SemaphoreType.DMA,
        pltpu.SemaphoreType.DMA,
        pltpu.SemaphoreType.DMA,
        pltpu.SemaphoreType.DMA,
    ],
  )
  def sc_agg(tu_hbm, ti_hbm,
             src0_hbm, dst0_hbm, src1_hbm, dst1_hbm,
             aggu_hbm, cntu_hbm, aggi_hbm, cnti_hbm,
             acc_sp, cnt_sp, srcv, dstv, rows, onesv,
             sem_g, sem_s, sem_c, sem_i):
    c = lax.axis_index("c")
    s = lax.axis_index("s")
    w = c * NS + s

    def _fill_narrow(value):
        def f(i, carry):
            onesv[i, pl.ds(0, CW)] = jnp.full((CW,), value, jnp.bfloat16)
            return carry

        lax.fori_loop(0, C, f, 0)

    row0 = s * RPT
    nfull = RPT // C
    rem = RPT % C

    def _fill_rows0(value):
        def f(i, carry):
            rows[0, i // 4, pl.ds((i % 4) * 32, 32)] = jnp.full(
                (32,), value, jnp.bfloat16)
            return carry

        lax.fori_loop(0, C * 4, f, 0)

    def _zero_stripes():
        _fill_rows0(0.0)
        _fill_narrow(0.0)
        for q in range(nfull):
            pltpu.sync_copy(rows.at[0], acc_sp.at[pl.ds(row0 + q * C, C)])
            pltpu.sync_copy(onesv, cnt_sp.at[pl.ds(row0 + q * C, C)])
        if rem:
            pltpu.sync_copy(rows.at[0, pl.ds(0, rem)],
                            acc_sp.at[pl.ds(row0 + nfull * C, rem)])
            pltpu.sync_copy(onesv.at[pl.ds(0, rem)],
                            cnt_sp.at[pl.ds(row0 + nfull * C, rem)])
        _fill_narrow(1.0)

    def _retire_row_scatter():
        pltpu.make_async_copy(rows.at[0], acc_sp.at[dstv.at[0]],
                              sem_s).wait()

    def _retire_ones_scatter():
        pltpu.make_async_copy(onesv, cnt_sp.at[dstv.at[0]],
                              sem_c).wait()

    def _stage_idx(idx_hbm, buf, g):
        pltpu.async_copy(idx_hbm.at[w, pl.ds(g * GC, GC)], buf, sem_i)

    def _wait_idx(buf):
        pltpu.make_async_copy(src0_hbm.at[0, pl.ds(0, GC)], buf,
                              sem_i).wait()

    def _edges(table_ref, src_hbm, dst_hbm):
        for g in range(NG):
            _stage_idx(src_hbm, srcv, g)
            _stage_idx(dst_hbm, dstv, g)
            _wait_idx(srcv)
            _wait_idx(dstv)
            pltpu.async_copy(table_ref.at[srcv.at[0]], rows.at[0], sem_g)

            def body(jj, carry):
                b = lax.rem(jj, 2)
                pltpu.make_async_copy(table_ref.at[srcv.at[jj]],
                                      rows.at[b], sem_g).wait()
                pltpu.async_copy(rows.at[b], acc_sp.at[dstv.at[jj]],
                                 sem_s, add=True)
                pltpu.async_copy(onesv, cnt_sp.at[dstv.at[jj]],
                                 sem_c, add=True)

                @pl.when(jj >= 1)
                def _():
                    _retire_row_scatter()

                @pl.when(jj >= 2)
                def _():
                    _retire_ones_scatter()

                @pl.when(jj + 1 < GC)
                def _():
                    pltpu.async_copy(table_ref.at[srcv.at[jj + 1]],
                                     rows.at[1 - b], sem_g)

                return carry

            lax.fori_loop(0, GC, body, 0)
            _retire_row_scatter()
            _retire_ones_scatter()
            _retire_ones_scatter()

    for r in range(2):
        table = tu_hbm if r == 0 else ti_hbm
        src_hbm = src0_hbm if r == 0 else src1_hbm
        dst_hbm = dst0_hbm if r == 0 else dst1_hbm
        agg_hbm = aggi_hbm if r == 0 else aggu_hbm
        cnt_hbm = cnti_hbm if r == 0 else cntu_hbm

        _zero_stripes()
        plsc.subcore_barrier()
        _edges(table, src_hbm, dst_hbm)
        plsc.subcore_barrier()

        @pl.when(c == 0)
        def _(agg=agg_hbm, cnt=cnt_hbm):
            pltpu.sync_copy(acc_sp.at[pl.ds(row0, RPT)],
                            agg.at[0, pl.ds(row0, RPT)])
            pltpu.sync_copy(cnt_sp.at[pl.ds(row0, RPT)],
                            cnt.at[0, pl.ds(row0, RPT)])

        @pl.when(c == 1)
        def _(agg=agg_hbm, cnt=cnt_hbm):
            pltpu.sync_copy(acc_sp.at[pl.ds(row0, RPT)],
                            agg.at[1, pl.ds(row0, RPT)])
            pltpu.sync_copy(cnt_sp.at[pl.ds(row0, RPT)],
                            cnt.at[1, pl.ds(row0, RPT)])

  return sc_agg


def _sc_agg(*args):
    return _build_sc_agg()(*args)


def _post_body(x_ref, s_ref, agg_ref, cnt_ref, g_ref, b_ref, w_ref, bm_ref,
               o_ref):
    x = x_ref[...]
    agg = (agg_ref[0].astype(jnp.float32) + agg_ref[1].astype(jnp.float32))
    cnt = (cnt_ref[0][:, 0:1].astype(jnp.float32) +
           cnt_ref[1][:, 0:1].astype(jnp.float32))
    x2 = x + s_ref[...] + agg / jnp.maximum(cnt, 1.0)
    mu = jnp.mean(x2, axis=-1, keepdims=True)
    var = jnp.mean((x2 - mu) ** 2, axis=-1, keepdims=True)
    z = jnp.maximum((x2 - mu) * lax.rsqrt(var + 1e-5) * g_ref[0] + b_ref[0],
                    0.0)
    o_ref[...] = x2 + jnp.dot(z, w_ref[...],
                              preferred_element_type=jnp.float32,
                              precision=lax.Precision.HIGHEST) + bm_ref[0]


def _post(x, sv, agg, cnt, g, b, w, bm):
    return pl.pallas_call(
        _post_body,
        grid=(NB,),
        in_specs=[
            pl.BlockSpec((BR, D), lambda i: (i, 0)),
            pl.BlockSpec((BR, D), lambda i: (i, 0)),
            pl.BlockSpec((2, BR, D), lambda i: (0, i, 0)),
            pl.BlockSpec((2, BR, CW), lambda i: (0, i, 0)),
            pl.BlockSpec((1, D), lambda i: (0, 0)),
            pl.BlockSpec((1, D), lambda i: (0, 0)),
            pl.BlockSpec((D, D), lambda i: (0, 0)),
            pl.BlockSpec((1, D), lambda i: (0, 0)),
        ],
        out_specs=pl.BlockSpec((BR, D), lambda i: (i, 0)),
        out_shape=jax.ShapeDtypeStruct((N, D), jnp.float32),
    )(x, sv, agg, cnt, g, b, w, bm)


def _pad_edges(ei):
    pad = EPAD - E
    ar = jnp.arange(pad, dtype=jnp.int32)
    src = jnp.concatenate([ei[0], ar % N])
    dst = jnp.concatenate([ei[1], N + ar % (NPAD - N)])
    return src.reshape(NW, K, C), dst.reshape(NW, K, C)


def kernel(x_user, x_item, ln1_g_user, ln1_b_user, ln1_g_item, ln1_b_item,
           W_self_user, W_self_item, W_u2i, W_i2u,
           ln2_g_user, ln2_b_user, ln2_g_item, ln2_b_item,
           W_mlp_user, b_mlp_user, W_mlp_item, b_mlp_item,
           edge_index_u2i, edge_index_i2u):
    tu, s_u = _prep(x_user, ln1_g_user.reshape(1, D),
                    ln1_b_user.reshape(1, D), W_u2i, W_self_user)
    ti, s_i = _prep(x_item, ln1_g_item.reshape(1, D),
                    ln1_b_item.reshape(1, D), W_i2u, W_self_item)

    src0, dst0 = _pad_edges(edge_index_u2i)
    src1, dst1 = _pad_edges(edge_index_i2u)

    agg_u, cnt_u, agg_i, cnt_i = _sc_agg(tu, ti, src0, dst0, src1, dst1)

    out_u = _post(x_user, s_u, agg_u, cnt_u, ln2_g_user.reshape(1, D),
                  ln2_b_user.reshape(1, D), W_mlp_user,
                  b_mlp_user.reshape(1, D))
    out_i = _post(x_item, s_i, agg_i, cnt_i, ln2_g_item.reshape(1, D),
                  ln2_b_item.reshape(1, D), W_mlp_item,
                  b_mlp_item.reshape(1, D))
    return (out_u, out_i)


# speedup vs baseline: 1.1297x; 1.1297x over previous
import functools

import jax
import jax.numpy as jnp
from jax import lax
from jax.experimental import pallas as pl
from jax.experimental.pallas import tpu as pltpu
from jax.experimental.pallas import tpu_sc as plsc

N = 25000
D = 128
E = 300000
NS = 16
NW = 32
C = 64
K = 147
EPAD = NW * C * K
NPAD = 25088
RPT = NPAD // NS
NB = 16
BR = NPAD // NB
NG = 21
GC = K // NG
CW = 32

assert K == NG * GC and EPAD >= E


def _prep_body(x_ref, g_ref, b_ref, wrel_ref, wself_ref, t_ref, s_ref):
    x = x_ref[...]
    mu = jnp.mean(x, axis=-1, keepdims=True)
    var = jnp.mean((x - mu) ** 2, axis=-1, keepdims=True)
    y = jnp.maximum((x - mu) * lax.rsqrt(var + 1e-5) * g_ref[0] + b_ref[0], 0.0)
    t_ref[...] = jnp.dot(y, wrel_ref[...], preferred_element_type=jnp.float32,
                         precision=lax.Precision.HIGHEST).astype(jnp.bfloat16)
    s_ref[...] = jnp.dot(y, wself_ref[...],
                         preferred_element_type=jnp.float32,
                         precision=lax.Precision.HIGHEST)


def _prep(x, g, b, wrel, wself):
    return pl.pallas_call(
        _prep_body,
        grid=(NB,),
        in_specs=[
            pl.BlockSpec((BR, D), lambda i: (i, 0)),
            pl.BlockSpec((1, D), lambda i: (0, 0)),
            pl.BlockSpec((1, D), lambda i: (0, 0)),
            pl.BlockSpec((D, D), lambda i: (0, 0)),
            pl.BlockSpec((D, D), lambda i: (0, 0)),
        ],
        out_specs=[
            pl.BlockSpec((BR, D), lambda i: (i, 0)),
            pl.BlockSpec((BR, D), lambda i: (i, 0)),
        ],
        out_shape=[jax.ShapeDtypeStruct((NPAD, D), jnp.bfloat16),
                   jax.ShapeDtypeStruct((N, D), jnp.float32)],
    )(x, g, b, wrel, wself)


@functools.cache
def _build_sc_agg():
  mesh = plsc.VectorSubcoreMesh(core_axis_name="c", subcore_axis_name="s")

  @functools.partial(
    pl.kernel,
    out_type=[
        jax.ShapeDtypeStruct((2, NPAD, D), jnp.bfloat16),
        jax.ShapeDtypeStruct((2, NPAD, D), jnp.bfloat16),
        jax.ShapeDtypeStruct((2, NPAD, D), jnp.bfloat16),
        jax.ShapeDtypeStruct((2, NPAD, D), jnp.bfloat16),
    ],
    mesh=mesh,
    compiler_params=pltpu.CompilerParams(use_tc_tiling_on_sc=False),
    scratch_types=[
        pltpu.VMEM_SHARED((NPAD, D), jnp.bfloat16),
        pltpu.VMEM((2, GC, C), jnp.int32),
        pltpu.VMEM((2, GC, C), jnp.int32),
        pltpu.VMEM((3, C, D), jnp.bfloat16),
        pltpu.SemaphoreType.DMA,
        pltpu.SemaphoreType.DMA,
        pltpu.SemaphoreType.DMA,
    ],
  )
  def sc_agg(tu_hbm, ti_hbm,
             src0_hbm, dst0_hbm, src1_hbm, dst1_hbm,
             aggu_hbm, cntu_hbm, aggi_hbm, cnti_hbm,
             acc_sp, srcv, dstv, rows,
             sem_g, sem_s, sem_i):
    c = lax.axis_index("c")
    s = lax.axis_index("s")
    w = c * NS + s

    row0 = s * RPT
    nfull = RPT // C
    rem = RPT % C

    def _fill_rows0(value):
        def f(i, carry):
            rows[0, i // 4, pl.ds((i % 4) * 32, 32)] = jnp.full(
                (32,), value, jnp.bfloat16)
            return carry

        lax.fori_loop(0, C * 4, f, 0)

    def _zero_stripe():
        _fill_rows0(0.0)
        for q in range(nfull):
            pltpu.sync_copy(rows.at[0], acc_sp.at[pl.ds(row0 + q * C, C)])
        if rem:
            pltpu.sync_copy(rows.at[0, pl.ds(0, rem)],
                            acc_sp.at[pl.ds(row0 + nfull * C, rem)])

    def _retire_scatter():
        pltpu.make_async_copy(rows.at[0], acc_sp.at[dstv.at[0, 0]],
                              sem_s).wait()

    def _stage_idx(idx_hbm, buf, g):
        pltpu.async_copy(idx_hbm.at[w, pl.ds(g * GC, GC)], buf, sem_i)

    def _wait_idx(buf):
        pltpu.make_async_copy(src0_hbm.at[0, pl.ds(0, GC)], buf,
                              sem_i).wait()

    def _edges(table_ref, src_hbm, dst_hbm):
        _stage_idx(src_hbm, srcv.at[0], 0)
        _stage_idx(dst_hbm, dstv.at[0], 0)
        for g in range(NG):
            gb = g % 2
            _wait_idx(srcv.at[gb])
            _wait_idx(dstv.at[gb])
            if g + 1 < NG:
                _stage_idx(src_hbm, srcv.at[(g + 1) % 2], g + 1)
                _stage_idx(dst_hbm, dstv.at[(g + 1) % 2], g + 1)
            pltpu.async_copy(table_ref.at[srcv.at[gb, 0]], rows.at[0], sem_g)

            pltpu.async_copy(table_ref.at[srcv.at[gb, 1]], rows.at[1],
                             sem_g)

            def body(jj, carry, gb=gb):
                b = lax.rem(jj, 3)
                pltpu.make_async_copy(table_ref.at[srcv.at[gb, jj]],
                                      rows.at[b], sem_g).wait()
                pltpu.async_copy(rows.at[b], acc_sp.at[dstv.at[gb, jj]],
                                 sem_s, add=True)

                @pl.when(jj >= 1)
                def _():
                    _retire_scatter()

                @pl.when(jj + 2 < GC)
                def _():
                    pltpu.async_copy(table_ref.at[srcv.at[gb, jj + 2]],
                                     rows.at[lax.rem(jj + 2, 3)], sem_g)

                return carry

            lax.fori_loop(0, GC, body, 0)
            _retire_scatter()

    for r in range(2):
        table = tu_hbm if r == 0 else ti_hbm
        src_hbm = src0_hbm if r == 0 else src1_hbm
        dst_hbm = dst0_hbm if r == 0 else dst1_hbm
        agg_hbm = aggi_hbm if r == 0 else aggu_hbm
        cnt_hbm = cnti_hbm if r == 0 else cntu_hbm

        _zero_stripe()
        plsc.subcore_barrier()
        _edges(table, src_hbm, dst_hbm)
        plsc.subcore_barrier()

        @pl.when(c == 0)
        def _(agg=agg_hbm):
            pltpu.sync_copy(acc_sp.at[pl.ds(row0, RPT)],
                            agg.at[0, pl.ds(row0, RPT)])

        @pl.when(c == 1)
        def _(agg=agg_hbm):
            pltpu.sync_copy(acc_sp.at[pl.ds(row0, RPT)],
                            agg.at[1, pl.ds(row0, RPT)])

        _zero_stripe()
        _fill_rows0(1.0)
        plsc.subcore_barrier()
        for g in range(NG):
            _stage_idx(dst_hbm, dstv.at[g % 2], g)
            _wait_idx(dstv.at[g % 2])

            def cbody(jj, carry, gb=g % 2):
                pltpu.async_copy(rows.at[0], acc_sp.at[dstv.at[gb, jj]],
                                 sem_s, add=True)

                @pl.when(jj >= 2)
                def _():
                    _retire_scatter()

                return carry

            lax.fori_loop(0, GC, cbody, 0)
            _retire_scatter()
            _retire_scatter()
        plsc.subcore_barrier()

        @pl.when(c == 0)
        def _(cnt=cnt_hbm):
            pltpu.sync_copy(acc_sp.at[pl.ds(row0, RPT)],
                            cnt.at[0, pl.ds(row0, RPT)])

        @pl.when(c == 1)
        def _(cnt=cnt_hbm):
            pltpu.sync_copy(acc_sp.at[pl.ds(row0, RPT)],
                            cnt.at[1, pl.ds(row0, RPT)])

  return sc_agg


def _sc_agg(*args):
    return _build_sc_agg()(*args)


def _post_body(x_ref, s_ref, agg_ref, cnt_ref, g_ref, b_ref, w_ref, bm_ref,
               o_ref):
    x = x_ref[...]
    agg = (agg_ref[0].astype(jnp.float32) + agg_ref[1].astype(jnp.float32))
    cnt = (cnt_ref[0][:, 0:1].astype(jnp.float32) +
           cnt_ref[1][:, 0:1].astype(jnp.float32))
    x2 = x + s_ref[...] + agg / jnp.maximum(cnt, 1.0)
    mu = jnp.mean(x2, axis=-1, keepdims=True)
    var = jnp.mean((x2 - mu) ** 2, axis=-1, keepdims=True)
    z = jnp.maximum((x2 - mu) * lax.rsqrt(var + 1e-5) * g_ref[0] + b_ref[0],
                    0.0)
    o_ref[...] = x2 + jnp.dot(z, w_ref[...],
                              preferred_element_type=jnp.float32,
                              precision=lax.Precision.HIGHEST) + bm_ref[0]


def _post(x, sv, agg, cnt, g, b, w, bm):
    return pl.pallas_call(
        _post_body,
        grid=(NB,),
        in_specs=[
            pl.BlockSpec((BR, D), lambda i: (i, 0)),
            pl.BlockSpec((BR, D), lambda i: (i, 0)),
            pl.BlockSpec((2, BR, D), lambda i: (0, i, 0)),
            pl.BlockSpec((2, BR, D), lambda i: (0, i, 0)),
            pl.BlockSpec((1, D), lambda i: (0, 0)),
            pl.BlockSpec((1, D), lambda i: (0, 0)),
            pl.BlockSpec((D, D), lambda i: (0, 0)),
            pl.BlockSpec((1, D), lambda i: (0, 0)),
        ],
        out_specs=pl.BlockSpec((BR, D), lambda i: (i, 0)),
        out_shape=jax.ShapeDtypeStruct((N, D), jnp.float32),
    )(x, sv, agg, cnt, g, b, w, bm)


def _pad_edges(ei):
    pad = EPAD - E
    ar = jnp.arange(pad, dtype=jnp.int32)
    src = jnp.concatenate([ei[0], ar % N])
    dst = jnp.concatenate([ei[1], N + ar % (NPAD - N)])
    return src.reshape(NW, K, C), dst.reshape(NW, K, C)


def kernel(x_user, x_item, ln1_g_user, ln1_b_user, ln1_g_item, ln1_b_item,
           W_self_user, W_self_item, W_u2i, W_i2u,
           ln2_g_user, ln2_b_user, ln2_g_item, ln2_b_item,
           W_mlp_user, b_mlp_user, W_mlp_item, b_mlp_item,
           edge_index_u2i, edge_index_i2u):
    tu, s_u = _prep(x_user, ln1_g_user.reshape(1, D),
                    ln1_b_user.reshape(1, D), W_u2i, W_self_user)
    ti, s_i = _prep(x_item, ln1_g_item.reshape(1, D),
                    ln1_b_item.reshape(1, D), W_i2u, W_self_item)

    src0, dst0 = _pad_edges(edge_index_u2i)
    src1, dst1 = _pad_edges(edge_index_i2u)

    agg_u, cnt_u, agg_i, cnt_i = _sc_agg(tu, ti, src0, dst0, src1, dst1)

    out_u = _post(x_user, s_u, agg_u, cnt_u, ln2_g_user.reshape(1, D),
                  ln2_b_user.reshape(1, D), W_mlp_user,
                  b_mlp_user.reshape(1, D))
    out_i = _post(x_item, s_i, agg_i, cnt_i, ln2_g_item.reshape(1, D),
                  ln2_b_item.reshape(1, D), W_mlp_item,
                  b_mlp_item.reshape(1, D))
    return (out_u, out_i)


# speedup vs baseline: 1.2659x; 1.1206x over previous
import functools

import jax
import jax.numpy as jnp
from jax import lax
from jax.experimental import pallas as pl
from jax.experimental.pallas import tpu as pltpu
from jax.experimental.pallas import tpu_sc as plsc

N = 25000
D = 128
E = 300000
HW = 64
NS = 16
C = 64
K = -(-E // (NS * C)) + 1
EPAD = NS * C * K
NPAD = 25088
RPT = NPAD // NS
NB = 16
BR = NPAD // NB
NG = 7
GC = K // NG
CW = 32

assert K == NG * GC and EPAD >= E


def _prep_body(x_ref, g_ref, b_ref, wrel_ref, wself_ref,
               t0_ref, t1_ref, s_ref):
    x = x_ref[...]
    mu = jnp.mean(x, axis=-1, keepdims=True)
    var = jnp.mean((x - mu) ** 2, axis=-1, keepdims=True)
    y = jnp.maximum((x - mu) * lax.rsqrt(var + 1e-5) * g_ref[0] + b_ref[0], 0.0)
    t = jnp.dot(y, wrel_ref[...], preferred_element_type=jnp.float32,
                precision=lax.Precision.HIGHEST).astype(jnp.bfloat16)
    t0_ref[...] = t[:, :HW]
    t1_ref[...] = t[:, HW:]
    s_ref[...] = jnp.dot(y, wself_ref[...],
                         preferred_element_type=jnp.float32,
                         precision=lax.Precision.HIGHEST)


def _prep(x, g, b, wrel, wself):
    th = jax.ShapeDtypeStruct((NPAD, HW), jnp.bfloat16)
    return pl.pallas_call(
        _prep_body,
        grid=(NB,),
        in_specs=[
            pl.BlockSpec((BR, D), lambda i: (i, 0)),
            pl.BlockSpec((1, D), lambda i: (0, 0)),
            pl.BlockSpec((1, D), lambda i: (0, 0)),
            pl.BlockSpec((D, D), lambda i: (0, 0)),
            pl.BlockSpec((D, D), lambda i: (0, 0)),
        ],
        out_specs=[
            pl.BlockSpec((BR, HW), lambda i: (i, 0)),
            pl.BlockSpec((BR, HW), lambda i: (i, 0)),
            pl.BlockSpec((BR, D), lambda i: (i, 0)),
        ],
        out_shape=[th, th, jax.ShapeDtypeStruct((N, D), jnp.float32)],
    )(x, g, b, wrel, wself)


@functools.cache
def _build_sc_agg():
  mesh = plsc.VectorSubcoreMesh(core_axis_name="c", subcore_axis_name="s")

  NBUF = 4
  LOOK = 2

  @functools.partial(
    pl.kernel,
    out_type=[
        jax.ShapeDtypeStruct((2, NPAD, HW), jnp.bfloat16),
        jax.ShapeDtypeStruct((NPAD, CW), jnp.bfloat16),
        jax.ShapeDtypeStruct((2, NPAD, HW), jnp.bfloat16),
        jax.ShapeDtypeStruct((NPAD, CW), jnp.bfloat16),
    ],
    mesh=mesh,
    compiler_params=pltpu.CompilerParams(use_tc_tiling_on_sc=False),
    scratch_types=[
        pltpu.VMEM_SHARED((NPAD, HW), jnp.bfloat16),
        pltpu.VMEM_SHARED((NPAD, CW), jnp.bfloat16),
        pltpu.VMEM((2, GC, C), jnp.int32),
        pltpu.VMEM((2, GC, C), jnp.int32),
        pltpu.VMEM((NBUF, C, HW), jnp.bfloat16),
        pltpu.VMEM((C, CW), jnp.bfloat16),
        pltpu.SemaphoreType.DMA,
        pltpu.SemaphoreType.DMA,
        pltpu.SemaphoreType.DMA,
        pltpu.SemaphoreType.DMA,
    ],
  )
  def sc_agg(tu0_hbm, tu1_hbm, ti0_hbm, ti1_hbm,
             src0_hbm, dst0_hbm, src1_hbm, dst1_hbm,
             aggu_hbm, cntu_hbm, aggi_hbm, cnti_hbm,
             acc_sp, cnt_sp, srcv, dstv, rows, onesv,
             sem_g, sem_s, sem_c, sem_i):
    c = lax.axis_index("c")
    s = lax.axis_index("s")

    row0 = s * RPT
    nfull = RPT // C
    rem = RPT % C

    def _fill_rows0(value):
        def f(i, carry):
            rows[0, i // 2, pl.ds((i % 2) * 32, 32)] = jnp.full(
                (32,), value, jnp.bfloat16)
            return carry

        lax.fori_loop(0, C * 2, f, 0)

    def _fill_narrow(value):
        def f(i, carry):
            onesv[i, pl.ds(0, CW)] = jnp.full((CW,), value, jnp.bfloat16)
            return carry

        lax.fori_loop(0, C, f, 0)

    def _zero_stripe(zero_cnt):
        _fill_rows0(0.0)
        if zero_cnt:
            _fill_narrow(0.0)
        for q in range(nfull):
            pltpu.sync_copy(rows.at[0], acc_sp.at[pl.ds(row0 + q * C, C)])
            if zero_cnt:
                pltpu.sync_copy(onesv, cnt_sp.at[pl.ds(row0 + q * C, C)])
        if rem:
            pltpu.sync_copy(rows.at[0, pl.ds(0, rem)],
                            acc_sp.at[pl.ds(row0 + nfull * C, rem)])
            if zero_cnt:
                pltpu.sync_copy(onesv.at[pl.ds(0, rem)],
                                cnt_sp.at[pl.ds(row0 + nfull * C, rem)])
        if zero_cnt:
            _fill_narrow(1.0)

    def _retire_scatter():
        pltpu.make_async_copy(rows.at[0], acc_sp.at[dstv.at[0, 0]],
                              sem_s).wait()

    def _retire_ones():
        pltpu.make_async_copy(onesv, cnt_sp.at[dstv.at[0, 0]],
                              sem_c).wait()

    def _stage_idx(idx_hbm, buf, g):
        pltpu.async_copy(idx_hbm.at[s, pl.ds(g * GC, GC)], buf, sem_i)

    def _wait_idx(buf):
        pltpu.make_async_copy(src0_hbm.at[s, pl.ds(0, GC)], buf,
                              sem_i).wait()

    def _edges(table_ref, src_hbm, dst_hbm, do_cnt):
        _stage_idx(src_hbm, srcv.at[0], 0)
        _stage_idx(dst_hbm, dstv.at[0], 0)
        for g in range(NG):
            gb = g % 2
            _wait_idx(srcv.at[gb])
            _wait_idx(dstv.at[gb])
            if g + 1 < NG:
                _stage_idx(src_hbm, srcv.at[(g + 1) % 2], g + 1)
                _stage_idx(dst_hbm, dstv.at[(g + 1) % 2], g + 1)
            for b in range(LOOK):
                pltpu.async_copy(table_ref.at[srcv.at[gb, b]], rows.at[b],
                                 sem_g)

            def body(jj, carry, gb=gb):
                b = lax.rem(jj, NBUF)
                pltpu.make_async_copy(table_ref.at[srcv.at[gb, jj]],
                                      rows.at[b], sem_g).wait()
                pltpu.async_copy(rows.at[b], acc_sp.at[dstv.at[gb, jj]],
                                 sem_s, add=True)
                if do_cnt:
                    pltpu.async_copy(onesv, cnt_sp.at[dstv.at[gb, jj]],
                                     sem_c, add=True)

                    @pl.when(jj >= 2)
                    def _():
                        _retire_ones()

                @pl.when(jj >= NBUF - LOOK)
                def _():
                    _retire_scatter()

                @pl.when(jj + LOOK < GC)
                def _():
                    pltpu.async_copy(table_ref.at[srcv.at[gb, jj + LOOK]],
                                     rows.at[lax.rem(jj + LOOK, NBUF)],
                                     sem_g)

                return carry

            lax.fori_loop(0, GC, body, 0)
            for _ in range(NBUF - LOOK):
                _retire_scatter()
            if do_cnt:
                _retire_ones()
                _retire_ones()

    for r in range(2):
        tabs = (tu0_hbm, tu1_hbm) if r == 0 else (ti0_hbm, ti1_hbm)
        src_hbm = src0_hbm if r == 0 else src1_hbm
        dst_hbm = dst0_hbm if r == 0 else dst1_hbm
        agg_hbm = aggi_hbm if r == 0 else aggu_hbm
        cnt_hbm = cnti_hbm if r == 0 else cntu_hbm

        _zero_stripe(zero_cnt=True)
        plsc.subcore_barrier()

        @pl.when(c == 0)
        def _(tab=tabs[0], sr=src_hbm, dr=dst_hbm, dc=(r == 0)):
            _edges(tab, sr, dr, dc)

        @pl.when(c == 1)
        def _(tab=tabs[1], sr=src_hbm, dr=dst_hbm, dc=(r == 1)):
            _edges(tab, sr, dr, dc)

        plsc.subcore_barrier()

        @pl.when(c == 0)
        def _(agg=agg_hbm, cnt=cnt_hbm, dc=(r == 0)):
            pltpu.sync_copy(acc_sp.at[pl.ds(row0, RPT)],
                            agg.at[0, pl.ds(row0, RPT)])
            if dc:
                pltpu.sync_copy(cnt_sp.at[pl.ds(row0, RPT)],
                                cnt.at[pl.ds(row0, RPT)])

        @pl.when(c == 1)
        def _(agg=agg_hbm, cnt=cnt_hbm, dc=(r == 1)):
            pltpu.sync_copy(acc_sp.at[pl.ds(row0, RPT)],
                            agg.at[1, pl.ds(row0, RPT)])
            if dc:
                pltpu.sync_copy(cnt_sp.at[pl.ds(row0, RPT)],
                                cnt.at[pl.ds(row0, RPT)])

  return sc_agg


def _sc_agg(*args):
    return _build_sc_agg()(*args)


def _post_body(x_ref, s_ref, agg_ref, cnt_ref, g_ref, b_ref, w_ref, bm_ref,
               o_ref):
    x = x_ref[...]
    agg = jnp.concatenate([agg_ref[0], agg_ref[1]],
                          axis=-1).astype(jnp.float32)
    cnt = jnp.maximum(cnt_ref[...][:, 0:1].astype(jnp.float32), 1.0)
    x2 = x + s_ref[...] + agg / cnt
    mu = jnp.mean(x2, axis=-1, keepdims=True)
    var = jnp.mean((x2 - mu) ** 2, axis=-1, keepdims=True)
    z = jnp.maximum((x2 - mu) * lax.rsqrt(var + 1e-5) * g_ref[0] + b_ref[0],
                    0.0)
    o_ref[...] = x2 + jnp.dot(z, w_ref[...],
                              preferred_element_type=jnp.float32,
                              precision=lax.Precision.HIGHEST) + bm_ref[0]


def _post(x, sv, agg, cnt, g, b, w, bm):
    return pl.pallas_call(
        _post_body,
        grid=(NB,),
        in_specs=[
            pl.BlockSpec((BR, D), lambda i: (i, 0)),
            pl.BlockSpec((BR, D), lambda i: (i, 0)),
            pl.BlockSpec((2, BR, HW), lambda i: (0, i, 0)),
            pl.BlockSpec((BR, CW), lambda i: (i, 0)),
            pl.BlockSpec((1, D), lambda i: (0, 0)),
            pl.BlockSpec((1, D), lambda i: (0, 0)),
            pl.BlockSpec((D, D), lambda i: (0, 0)),
            pl.BlockSpec((1, D), lambda i: (0, 0)),
        ],
        out_specs=pl.BlockSpec((BR, D), lambda i: (i, 0)),
        out_shape=jax.ShapeDtypeStruct((N, D), jnp.float32),
    )(x, sv, agg, cnt, g, b, w, bm)


def _pad_edges(ei):
    pad = EPAD - E
    ar = jnp.arange(pad, dtype=jnp.int32)
    src = jnp.concatenate([ei[0], ar % N])
    dst = jnp.concatenate([ei[1], N + ar % (NPAD - N)])
    return src.reshape(NS, K, C), dst.reshape(NS, K, C)


def kernel(x_user, x_item, ln1_g_user, ln1_b_user, ln1_g_item, ln1_b_item,
           W_self_user, W_self_item, W_u2i, W_i2u,
           ln2_g_user, ln2_b_user, ln2_g_item, ln2_b_item,
           W_mlp_user, b_mlp_user, W_mlp_item, b_mlp_item,
           edge_index_u2i, edge_index_i2u):
    tu = _prep(x_user, ln1_g_user.reshape(1, D), ln1_b_user.reshape(1, D),
               W_u2i, W_self_user)
    ti = _prep(x_item, ln1_g_item.reshape(1, D), ln1_b_item.reshape(1, D),
               W_i2u, W_self_item)
    s_u, s_i = tu[2], ti[2]

    src0, dst0 = _pad_edges(edge_index_u2i)
    src1, dst1 = _pad_edges(edge_index_i2u)

    agg_u, cnt_u, agg_i, cnt_i = _sc_agg(tu[0], tu[1], ti[0], ti[1],
                                         src0, dst0, src1, dst1)

    out_u = _post(x_user, s_u, agg_u, cnt_u, ln2_g_user.reshape(1, D),
                  ln2_b_user.reshape(1, D), W_mlp_user,
                  b_mlp_user.reshape(1, D))
    out_i = _post(x_item, s_i, agg_i, cnt_i, ln2_g_item.reshape(1, D),
                  ln2_b_item.reshape(1, D), W_mlp_item,
                  b_mlp_item.reshape(1, D))
    return (out_u, out_i)


# speedup vs baseline: 1.4346x; 1.1333x over previous
import functools

import jax
import jax.numpy as jnp
from jax import lax
from jax.experimental import pallas as pl
from jax.experimental.pallas import tpu as pltpu
from jax.experimental.pallas import tpu_sc as plsc

N = 25000
D = 128
E = 300000
HW = 64
NS = 16
C = 128
K = -(-E // (NS * C))
EPAD = NS * C * K
NPAD = 25088
RPT = NPAD // NS
NB = 16
BR = NPAD // NB
NG = 7
GC = K // NG
CW = 32

assert K == NG * GC and EPAD >= E


def _prep_body(x_ref, g_ref, b_ref, wrel_ref, wself_ref,
               t0_ref, t1_ref, s_ref):
    x = x_ref[...]
    mu = jnp.mean(x, axis=-1, keepdims=True)
    var = jnp.mean((x - mu) ** 2, axis=-1, keepdims=True)
    y = jnp.maximum((x - mu) * lax.rsqrt(var + 1e-5) * g_ref[0] + b_ref[0], 0.0)
    t = jnp.dot(y, wrel_ref[...], preferred_element_type=jnp.float32,
                precision=lax.Precision.HIGHEST).astype(jnp.bfloat16)
    t0_ref[...] = t[:, :HW]
    t1_ref[...] = t[:, HW:]
    s_ref[...] = jnp.dot(y, wself_ref[...],
                         preferred_element_type=jnp.float32,
                         precision=lax.Precision.HIGHEST)


def _prep(x, g, b, wrel, wself):
    th = jax.ShapeDtypeStruct((NPAD, HW), jnp.bfloat16)
    return pl.pallas_call(
        _prep_body,
        grid=(NB,),
        in_specs=[
            pl.BlockSpec((BR, D), lambda i: (i, 0)),
            pl.BlockSpec((1, D), lambda i: (0, 0)),
            pl.BlockSpec((1, D), lambda i: (0, 0)),
            pl.BlockSpec((D, D), lambda i: (0, 0)),
            pl.BlockSpec((D, D), lambda i: (0, 0)),
        ],
        out_specs=[
            pl.BlockSpec((BR, HW), lambda i: (i, 0)),
            pl.BlockSpec((BR, HW), lambda i: (i, 0)),
            pl.BlockSpec((BR, D), lambda i: (i, 0)),
        ],
        out_shape=[th, th, jax.ShapeDtypeStruct((N, D), jnp.float32)],
    )(x, g, b, wrel, wself)


@functools.cache
def _build_sc_agg():
  mesh = plsc.VectorSubcoreMesh(core_axis_name="c", subcore_axis_name="s")

  NBUF = 4
  LOOK = 2

  @functools.partial(
    pl.kernel,
    out_type=[
        jax.ShapeDtypeStruct((2, NPAD, HW), jnp.bfloat16),
        jax.ShapeDtypeStruct((NPAD, CW), jnp.bfloat16),
        jax.ShapeDtypeStruct((2, NPAD, HW), jnp.bfloat16),
        jax.ShapeDtypeStruct((NPAD, CW), jnp.bfloat16),
    ],
    mesh=mesh,
    compiler_params=pltpu.CompilerParams(use_tc_tiling_on_sc=False),
    scratch_types=[
        pltpu.VMEM_SHARED((NPAD, HW), jnp.bfloat16),
        pltpu.VMEM_SHARED((NPAD, CW), jnp.bfloat16),
        pltpu.VMEM((2, GC, C), jnp.int32),
        pltpu.VMEM((2, GC, C), jnp.int32),
        pltpu.VMEM((NBUF, C, HW), jnp.bfloat16),
        pltpu.VMEM((C, CW), jnp.bfloat16),
        pltpu.SemaphoreType.DMA,
        pltpu.SemaphoreType.DMA,
        pltpu.SemaphoreType.DMA,
        pltpu.SemaphoreType.DMA,
    ],
  )
  def sc_agg(tu0_hbm, tu1_hbm, ti0_hbm, ti1_hbm,
             src0_hbm, dst0_hbm, src1_hbm, dst1_hbm,
             aggu_hbm, cntu_hbm, aggi_hbm, cnti_hbm,
             acc_sp, cnt_sp, srcv, dstv, rows, onesv,
             sem_g, sem_s, sem_c, sem_i):
    c = lax.axis_index("c")
    s = lax.axis_index("s")

    row0 = s * RPT
    nfull = RPT // C
    rem = RPT % C

    def _fill_rows0(value):
        def f(i, carry):
            rows[0, i // 2, pl.ds((i % 2) * 32, 32)] = jnp.full(
                (32,), value, jnp.bfloat16)
            return carry

        lax.fori_loop(0, C * 2, f, 0)

    def _fill_narrow(value):
        def f(i, carry):
            onesv[i, pl.ds(0, CW)] = jnp.full((CW,), value, jnp.bfloat16)
            return carry

        lax.fori_loop(0, C, f, 0)

    def _zero_stripe(zero_cnt):
        _fill_rows0(0.0)
        if zero_cnt:
            _fill_narrow(0.0)
        for q in range(nfull):
            pltpu.sync_copy(rows.at[0], acc_sp.at[pl.ds(row0 + q * C, C)])
            if zero_cnt:
                pltpu.sync_copy(onesv, cnt_sp.at[pl.ds(row0 + q * C, C)])
        if rem:
            pltpu.sync_copy(rows.at[0, pl.ds(0, rem)],
                            acc_sp.at[pl.ds(row0 + nfull * C, rem)])
            if zero_cnt:
                pltpu.sync_copy(onesv.at[pl.ds(0, rem)],
                                cnt_sp.at[pl.ds(row0 + nfull * C, rem)])
        if zero_cnt:
            _fill_narrow(1.0)

    def _retire_scatter():
        pltpu.make_async_copy(rows.at[0], acc_sp.at[dstv.at[0, 0]],
                              sem_s).wait()

    def _retire_ones():
        pltpu.make_async_copy(onesv, cnt_sp.at[dstv.at[0, 0]],
                              sem_c).wait()

    def _stage_idx(idx_hbm, buf, g):
        pltpu.async_copy(idx_hbm.at[s, pl.ds(g * GC, GC)], buf, sem_i)

    def _wait_idx(buf):
        pltpu.make_async_copy(src0_hbm.at[s, pl.ds(0, GC)], buf,
                              sem_i).wait()

    def _edges(table_ref, src_hbm, dst_hbm, do_cnt):
        _stage_idx(src_hbm, srcv.at[0], 0)
        _stage_idx(dst_hbm, dstv.at[0], 0)
        for g in range(NG):
            gb = g % 2
            _wait_idx(srcv.at[gb])
            _wait_idx(dstv.at[gb])
            if g + 1 < NG:
                _stage_idx(src_hbm, srcv.at[(g + 1) % 2], g + 1)
                _stage_idx(dst_hbm, dstv.at[(g + 1) % 2], g + 1)
            for b in range(LOOK):
                pltpu.async_copy(table_ref.at[srcv.at[gb, b]], rows.at[b],
                                 sem_g)

            def body(jj, carry, gb=gb):
                b = lax.rem(jj, NBUF)
                pltpu.make_async_copy(table_ref.at[srcv.at[gb, jj]],
                                      rows.at[b], sem_g).wait()
                pltpu.async_copy(rows.at[b], acc_sp.at[dstv.at[gb, jj]],
                                 sem_s, add=True)
                if do_cnt:
                    pltpu.async_copy(onesv, cnt_sp.at[dstv.at[gb, jj]],
                                     sem_c, add=True)

                    @pl.when(jj >= 2)
                    def _():
                        _retire_ones()

                @pl.when(jj >= NBUF - LOOK)
                def _():
                    _retire_scatter()

                @pl.when(jj + LOOK < GC)
                def _():
                    pltpu.async_copy(table_ref.at[srcv.at[gb, jj + LOOK]],
                                     rows.at[lax.rem(jj + LOOK, NBUF)],
                                     sem_g)

                return carry

            lax.fori_loop(0, GC, body, 0)
            for _ in range(NBUF - LOOK):
                _retire_scatter()
            if do_cnt:
                _retire_ones()
                _retire_ones()

    for r in range(2):
        tabs = (tu0_hbm, tu1_hbm) if r == 0 else (ti0_hbm, ti1_hbm)
        src_hbm = src0_hbm if r == 0 else src1_hbm
        dst_hbm = dst0_hbm if r == 0 else dst1_hbm
        agg_hbm = aggi_hbm if r == 0 else aggu_hbm
        cnt_hbm = cnti_hbm if r == 0 else cntu_hbm

        _zero_stripe(zero_cnt=True)
        plsc.subcore_barrier()

        @pl.when(c == 0)
        def _(tab=tabs[0], sr=src_hbm, dr=dst_hbm, dc=(r == 0)):
            _edges(tab, sr, dr, dc)

        @pl.when(c == 1)
        def _(tab=tabs[1], sr=src_hbm, dr=dst_hbm, dc=(r == 1)):
            _edges(tab, sr, dr, dc)

        plsc.subcore_barrier()

        @pl.when(c == 0)
        def _(agg=agg_hbm, cnt=cnt_hbm, dc=(r == 0)):
            pltpu.sync_copy(acc_sp.at[pl.ds(row0, RPT)],
                            agg.at[0, pl.ds(row0, RPT)])
            if dc:
                pltpu.sync_copy(cnt_sp.at[pl.ds(row0, RPT)],
                                cnt.at[pl.ds(row0, RPT)])

        @pl.when(c == 1)
        def _(agg=agg_hbm, cnt=cnt_hbm, dc=(r == 1)):
            pltpu.sync_copy(acc_sp.at[pl.ds(row0, RPT)],
                            agg.at[1, pl.ds(row0, RPT)])
            if dc:
                pltpu.sync_copy(cnt_sp.at[pl.ds(row0, RPT)],
                                cnt.at[pl.ds(row0, RPT)])

  return sc_agg


def _sc_agg(*args):
    return _build_sc_agg()(*args)


def _post_body(x_ref, s_ref, agg_ref, cnt_ref, g_ref, b_ref, w_ref, bm_ref,
               o_ref):
    x = x_ref[...]
    agg = jnp.concatenate([agg_ref[0], agg_ref[1]],
                          axis=-1).astype(jnp.float32)
    cnt = jnp.maximum(cnt_ref[...][:, 0:1].astype(jnp.float32), 1.0)
    x2 = x + s_ref[...] + agg / cnt
    mu = jnp.mean(x2, axis=-1, keepdims=True)
    var = jnp.mean((x2 - mu) ** 2, axis=-1, keepdims=True)
    z = jnp.maximum((x2 - mu) * lax.rsqrt(var + 1e-5) * g_ref[0] + b_ref[0],
                    0.0)
    o_ref[...] = x2 + jnp.dot(z, w_ref[...],
                              preferred_element_type=jnp.float32,
                              precision=lax.Precision.HIGHEST) + bm_ref[0]


def _post(x, sv, agg, cnt, g, b, w, bm):
    return pl.pallas_call(
        _post_body,
        grid=(NB,),
        in_specs=[
            pl.BlockSpec((BR, D), lambda i: (i, 0)),
            pl.BlockSpec((BR, D), lambda i: (i, 0)),
            pl.BlockSpec((2, BR, HW), lambda i: (0, i, 0)),
            pl.BlockSpec((BR, CW), lambda i: (i, 0)),
            pl.BlockSpec((1, D), lambda i: (0, 0)),
            pl.BlockSpec((1, D), lambda i: (0, 0)),
            pl.BlockSpec((D, D), lambda i: (0, 0)),
            pl.BlockSpec((1, D), lambda i: (0, 0)),
        ],
        out_specs=pl.BlockSpec((BR, D), lambda i: (i, 0)),
        out_shape=jax.ShapeDtypeStruct((N, D), jnp.float32),
    )(x, sv, agg, cnt, g, b, w, bm)


def _pad_edges(ei):
    pad = EPAD - E
    ar = jnp.arange(pad, dtype=jnp.int32)
    src = jnp.concatenate([ei[0], ar % N])
    dst = jnp.concatenate([ei[1], N + ar % (NPAD - N)])
    return src.reshape(NS, K, C), dst.reshape(NS, K, C)


def kernel(x_user, x_item, ln1_g_user, ln1_b_user, ln1_g_item, ln1_b_item,
           W_self_user, W_self_item, W_u2i, W_i2u,
           ln2_g_user, ln2_b_user, ln2_g_item, ln2_b_item,
           W_mlp_user, b_mlp_user, W_mlp_item, b_mlp_item,
           edge_index_u2i, edge_index_i2u):
    tu = _prep(x_user, ln1_g_user.reshape(1, D), ln1_b_user.reshape(1, D),
               W_u2i, W_self_user)
    ti = _prep(x_item, ln1_g_item.reshape(1, D), ln1_b_item.reshape(1, D),
               W_i2u, W_self_item)
    s_u, s_i = tu[2], ti[2]

    src0, dst0 = _pad_edges(edge_index_u2i)
    src1, dst1 = _pad_edges(edge_index_i2u)

    agg_u, cnt_u, agg_i, cnt_i = _sc_agg(tu[0], tu[1], ti[0], ti[1],
                                         src0, dst0, src1, dst1)

    out_u = _post(x_user, s_u, agg_u, cnt_u, ln2_g_user.reshape(1, D),
                  ln2_b_user.reshape(1, D), W_mlp_user,
                  b_mlp_user.reshape(1, D))
    out_i = _post(x_item, s_i, agg_i, cnt_i, ln2_g_item.reshape(1, D),
                  ln2_b_item.reshape(1, D), W_mlp_item,
                  b_mlp_item.reshape(1, D))
    return (out_u, out_i)


# speedup vs baseline: 1.5103x; 1.0528x over previous
import functools

import jax
import jax.numpy as jnp
from jax import lax
from jax.experimental import pallas as pl
from jax.experimental.pallas import tpu as pltpu
from jax.experimental.pallas import tpu_sc as plsc

N = 25000
D = 128
E = 300000
HW = 64
NS = 16
C = 128
K = -(-E // (NS * C))
EPAD = NS * C * K
NPAD = 25088
RPT = NPAD // NS
NB = 16
BR = NPAD // NB
NG = 7
GC = K // NG
CW = 32

assert K == NG * GC and EPAD >= E


def _prep_body(x_ref, g_ref, b_ref, wrel_ref, wself_ref,
               t0_ref, t1_ref, s_ref):
    x = x_ref[...]
    mu = jnp.mean(x, axis=-1, keepdims=True)
    var = jnp.mean((x - mu) ** 2, axis=-1, keepdims=True)
    y = jnp.maximum((x - mu) * lax.rsqrt(var + 1e-5) * g_ref[0] + b_ref[0], 0.0)
    t = jnp.dot(y, wrel_ref[...], preferred_element_type=jnp.float32,
                precision=lax.Precision.HIGHEST).astype(jnp.bfloat16)
    t0_ref[...] = t[:, :HW]
    t1_ref[...] = t[:, HW:]
    s_ref[...] = jnp.dot(y, wself_ref[...],
                         preferred_element_type=jnp.float32,
                         precision=lax.Precision.HIGHEST)


def _prep(x, g, b, wrel, wself):
    th = jax.ShapeDtypeStruct((NPAD, HW), jnp.bfloat16)
    return pl.pallas_call(
        _prep_body,
        grid=(NB,),
        in_specs=[
            pl.BlockSpec((BR, D), lambda i: (i, 0)),
            pl.BlockSpec((1, D), lambda i: (0, 0)),
            pl.BlockSpec((1, D), lambda i: (0, 0)),
            pl.BlockSpec((D, D), lambda i: (0, 0)),
            pl.BlockSpec((D, D), lambda i: (0, 0)),
        ],
        out_specs=[
            pl.BlockSpec((BR, HW), lambda i: (i, 0)),
            pl.BlockSpec((BR, HW), lambda i: (i, 0)),
            pl.BlockSpec((BR, D), lambda i: (i, 0)),
        ],
        out_shape=[th, th, jax.ShapeDtypeStruct((N, D), jnp.float32)],
    )(x, g, b, wrel, wself)


@functools.cache
def _build_sc_agg():
  mesh = plsc.VectorSubcoreMesh(core_axis_name="c", subcore_axis_name="s")

  NBUF = 6
  LOOK = 3

  @functools.partial(
    pl.kernel,
    out_type=[
        jax.ShapeDtypeStruct((2, NPAD, HW), jnp.bfloat16),
        jax.ShapeDtypeStruct((NPAD, CW), jnp.bfloat16),
        jax.ShapeDtypeStruct((2, NPAD, HW), jnp.bfloat16),
        jax.ShapeDtypeStruct((NPAD, CW), jnp.bfloat16),
    ],
    mesh=mesh,
    compiler_params=pltpu.CompilerParams(use_tc_tiling_on_sc=False),
    scratch_types=[
        pltpu.VMEM_SHARED((NPAD, HW), jnp.bfloat16),
        pltpu.VMEM_SHARED((NPAD, CW), jnp.bfloat16),
        pltpu.VMEM((2, GC, C), jnp.int32),
        pltpu.VMEM((2, GC, C), jnp.int32),
        pltpu.VMEM((NBUF, C, HW), jnp.bfloat16),
        pltpu.VMEM((C, CW), jnp.bfloat16),
        pltpu.SemaphoreType.DMA,
        pltpu.SemaphoreType.DMA,
        pltpu.SemaphoreType.DMA,
        pltpu.SemaphoreType.DMA,
    ],
  )
  def sc_agg(tu0_hbm, tu1_hbm, ti0_hbm, ti1_hbm,
             src0_hbm, dst0_hbm, src1_hbm, dst1_hbm,
             aggu_hbm, cntu_hbm, aggi_hbm, cnti_hbm,
             acc_sp, cnt_sp, srcv, dstv, rows, onesv,
             sem_g, sem_s, sem_c, sem_i):
    c = lax.axis_index("c")
    s = lax.axis_index("s")

    row0 = s * RPT
    nfull = RPT // C
    rem = RPT % C

    def _fill_rows0(value):
        def f(i, carry):
            rows[0, i // 2, pl.ds((i % 2) * 32, 32)] = jnp.full(
                (32,), value, jnp.bfloat16)
            return carry

        lax.fori_loop(0, C * 2, f, 0)

    def _fill_narrow(value):
        def f(i, carry):
            onesv[i, pl.ds(0, CW)] = jnp.full((CW,), value, jnp.bfloat16)
            return carry

        lax.fori_loop(0, C, f, 0)

    def _zero_stripe(zero_cnt):
        _fill_rows0(0.0)
        if zero_cnt:
            _fill_narrow(0.0)
        for q in range(nfull):
            pltpu.sync_copy(rows.at[0], acc_sp.at[pl.ds(row0 + q * C, C)])
            if zero_cnt:
                pltpu.sync_copy(onesv, cnt_sp.at[pl.ds(row0 + q * C, C)])
        if rem:
            pltpu.sync_copy(rows.at[0, pl.ds(0, rem)],
                            acc_sp.at[pl.ds(row0 + nfull * C, rem)])
            if zero_cnt:
                pltpu.sync_copy(onesv.at[pl.ds(0, rem)],
                                cnt_sp.at[pl.ds(row0 + nfull * C, rem)])
        if zero_cnt:
            _fill_narrow(1.0)

    def _retire_scatter():
        pltpu.make_async_copy(rows.at[0], acc_sp.at[dstv.at[0, 0]],
                              sem_s).wait()

    def _retire_ones():
        pltpu.make_async_copy(onesv, cnt_sp.at[dstv.at[0, 0]],
                              sem_c).wait()

    def _stage_idx(idx_hbm, buf, g):
        pltpu.async_copy(idx_hbm.at[s, pl.ds(g * GC, GC)], buf, sem_i)

    def _wait_idx(buf):
        pltpu.make_async_copy(src0_hbm.at[s, pl.ds(0, GC)], buf,
                              sem_i).wait()

    def _edges(table_ref, src_hbm, dst_hbm, do_cnt):
        _stage_idx(src_hbm, srcv.at[0], 0)
        _stage_idx(dst_hbm, dstv.at[0], 0)
        for g in range(NG):
            gb = g % 2
            _wait_idx(srcv.at[gb])
            _wait_idx(dstv.at[gb])
            if g + 1 < NG:
                _stage_idx(src_hbm, srcv.at[(g + 1) % 2], g + 1)
                _stage_idx(dst_hbm, dstv.at[(g + 1) % 2], g + 1)
            for b in range(LOOK):
                pltpu.async_copy(table_ref.at[srcv.at[gb, b]], rows.at[b],
                                 sem_g)

            def body(jj, carry, gb=gb):
                b = lax.rem(jj, NBUF)
                pltpu.make_async_copy(table_ref.at[srcv.at[gb, jj]],
                                      rows.at[b], sem_g).wait()
                pltpu.async_copy(rows.at[b], acc_sp.at[dstv.at[gb, jj]],
                                 sem_s, add=True)
                if do_cnt:
                    pltpu.async_copy(onesv, cnt_sp.at[dstv.at[gb, jj]],
                                     sem_c, add=True)

                    @pl.when(jj >= 2)
                    def _():
                        _retire_ones()

                @pl.when(jj >= NBUF - LOOK)
                def _():
                    _retire_scatter()

                @pl.when(jj + LOOK < GC)
                def _():
                    pltpu.async_copy(table_ref.at[srcv.at[gb, jj + LOOK]],
                                     rows.at[lax.rem(jj + LOOK, NBUF)],
                                     sem_g)

                return carry

            lax.fori_loop(0, GC, body, 0)
            for _ in range(NBUF - LOOK):
                _retire_scatter()
            if do_cnt:
                _retire_ones()
                _retire_ones()

    for r in range(2):
        tabs = (tu0_hbm, tu1_hbm) if r == 0 else (ti0_hbm, ti1_hbm)
        src_hbm = src0_hbm if r == 0 else src1_hbm
        dst_hbm = dst0_hbm if r == 0 else dst1_hbm
        agg_hbm = aggi_hbm if r == 0 else aggu_hbm
        cnt_hbm = cnti_hbm if r == 0 else cntu_hbm

        _zero_stripe(zero_cnt=True)
        plsc.subcore_barrier()

        @pl.when(c == 0)
        def _(tab=tabs[0], sr=src_hbm, dr=dst_hbm, dc=(r == 0)):
            _edges(tab, sr, dr, dc)

        @pl.when(c == 1)
        def _(tab=tabs[1], sr=src_hbm, dr=dst_hbm, dc=(r == 1)):
            _edges(tab, sr, dr, dc)

        plsc.subcore_barrier()

        @pl.when(c == 0)
        def _(agg=agg_hbm, cnt=cnt_hbm, dc=(r == 0)):
            pltpu.sync_copy(acc_sp.at[pl.ds(row0, RPT)],
                            agg.at[0, pl.ds(row0, RPT)])
            if dc:
                pltpu.sync_copy(cnt_sp.at[pl.ds(row0, RPT)],
                                cnt.at[pl.ds(row0, RPT)])

        @pl.when(c == 1)
        def _(agg=agg_hbm, cnt=cnt_hbm, dc=(r == 1)):
            pltpu.sync_copy(acc_sp.at[pl.ds(row0, RPT)],
                            agg.at[1, pl.ds(row0, RPT)])
            if dc:
                pltpu.sync_copy(cnt_sp.at[pl.ds(row0, RPT)],
                                cnt.at[pl.ds(row0, RPT)])

  return sc_agg


def _sc_agg(*args):
    return _build_sc_agg()(*args)


def _post_body(x_ref, s_ref, agg_ref, cnt_ref, g_ref, b_ref, w_ref, bm_ref,
               o_ref):
    x = x_ref[...]
    agg = jnp.concatenate([agg_ref[0], agg_ref[1]],
                          axis=-1).astype(jnp.float32)
    cnt = jnp.maximum(cnt_ref[...][:, 0:1].astype(jnp.float32), 1.0)
    x2 = x + s_ref[...] + agg / cnt
    mu = jnp.mean(x2, axis=-1, keepdims=True)
    var = jnp.mean((x2 - mu) ** 2, axis=-1, keepdims=True)
    z = jnp.maximum((x2 - mu) * lax.rsqrt(var + 1e-5) * g_ref[0] + b_ref[0],
                    0.0)
    o_ref[...] = x2 + jnp.dot(z, w_ref[...],
                              preferred_element_type=jnp.float32,
                              precision=lax.Precision.HIGHEST) + bm_ref[0]


def _post(x, sv, agg, cnt, g, b, w, bm):
    return pl.pallas_call(
        _post_body,
        grid=(NB,),
        in_specs=[
            pl.BlockSpec((BR, D), lambda i: (i, 0)),
            pl.BlockSpec((BR, D), lambda i: (i, 0)),
            pl.BlockSpec((2, BR, HW), lambda i: (0, i, 0)),
            pl.BlockSpec((BR, CW), lambda i: (i, 0)),
            pl.BlockSpec((1, D), lambda i: (0, 0)),
            pl.BlockSpec((1, D), lambda i: (0, 0)),
            pl.BlockSpec((D, D), lambda i: (0, 0)),
            pl.BlockSpec((1, D), lambda i: (0, 0)),
        ],
        out_specs=pl.BlockSpec((BR, D), lambda i: (i, 0)),
        out_shape=jax.ShapeDtypeStruct((N, D), jnp.float32),
    )(x, sv, agg, cnt, g, b, w, bm)


def _pad_edges(ei):
    pad = EPAD - E
    ar = jnp.arange(pad, dtype=jnp.int32)
    src = jnp.concatenate([ei[0], ar % N])
    dst = jnp.concatenate([ei[1], N + ar % (NPAD - N)])
    return src.reshape(NS, K, C), dst.reshape(NS, K, C)


def kernel(x_user, x_item, ln1_g_user, ln1_b_user, ln1_g_item, ln1_b_item,
           W_self_user, W_self_item, W_u2i, W_i2u,
           ln2_g_user, ln2_b_user, ln2_g_item, ln2_b_item,
           W_mlp_user, b_mlp_user, W_mlp_item, b_mlp_item,
           edge_index_u2i, edge_index_i2u):
    tu = _prep(x_user, ln1_g_user.reshape(1, D), ln1_b_user.reshape(1, D),
               W_u2i, W_self_user)
    ti = _prep(x_item, ln1_g_item.reshape(1, D), ln1_b_item.reshape(1, D),
               W_i2u, W_self_item)
    s_u, s_i = tu[2], ti[2]

    src0, dst0 = _pad_edges(edge_index_u2i)
    src1, dst1 = _pad_edges(edge_index_i2u)

    agg_u, cnt_u, agg_i, cnt_i = _sc_agg(tu[0], tu[1], ti[0], ti[1],
                                         src0, dst0, src1, dst1)

    out_u = _post(x_user, s_u, agg_u, cnt_u, ln2_g_user.reshape(1, D),
                  ln2_b_user.reshape(1, D), W_mlp_user,
                  b_mlp_user.reshape(1, D))
    out_i = _post(x_item, s_i, agg_i, cnt_i, ln2_g_item.reshape(1, D),
                  ln2_b_item.reshape(1, D), W_mlp_item,
                  b_mlp_item.reshape(1, D))
    return (out_u, out_i)
